# baseline (speedup 1.0000x reference)
import functools

import jax
import jax.numpy as jnp
from jax import lax
from jax.experimental import pallas as pl
from jax.experimental.pallas import tpu as pltpu
from jax.experimental.pallas import tpu_sc as plsc

VOCAB = 100000
EMBED_DIM = 64
HIDDEN = 128
L = 200

NQ = 4
VB = 1000
NB = 25
NR = NQ * NB


_NWRK = 25
_IPW = 8


def _lane_bcast(v, k):
    return lax.gather(
        v, jnp.full((16, 1), k),
        lax.GatherDimensionNumbers(
            offset_dims=(), collapsed_slice_dims=(0,),
            start_index_map=(0,)),
        (1,), mode=lax.GatherScatterMode.PROMISE_IN_BOUNDS)


@functools.cache
def _sc_bag_sum_kernel():
    mesh = plsc.VectorSubcoreMesh(core_axis_name="c", subcore_axis_name="s")
    return pl.kernel(
        _sc_bag_sum,
        out_type=jax.ShapeDtypeStruct((_NWRK, EMBED_DIM, 16), jnp.float32),
        mesh=mesh,
        scratch_types=[
            pltpu.VMEM((16,), jnp.int32),
            pltpu.VMEM((_IPW, EMBED_DIM, 128), jnp.float32),
            pltpu.VMEM((EMBED_DIM, 16), jnp.float32),
            pltpu.SemaphoreType.DMA,
        ],
    )


def _sc_bag_sum(idx_hbm, embT_hbm, out_hbm, idx_v, chunks_v, acc_v, sem):
    cid = lax.axis_index("c")
    sid = lax.axis_index("s")
    wid = sid * 2 + cid

    @pl.when(wid < _NWRK)
    def _():
        pltpu.sync_copy(idx_hbm.at[wid], idx_v)
        vrow = idx_v[...]
        cps = []
        for i in range(_IPW):
            col = pl.multiple_of((vrow[i] >> 7) << 7, 128)
            cp = pltpu.make_async_copy(
                embT_hbm.at[:, pl.ds(col, 128)], chunks_v.at[i], sem)
            cp.start()
            cps.append(cp)

        lane = lax.iota(jnp.int32, 16)
        pv = vrow & 127
        masks = [lane == (_lane_bcast(pv, i) & 15) for i in range(_IPW)]
        offs = [(pv[i] >> 4) << 4 for i in range(_IPW)]
        for cp in cps:
            cp.wait()

        def body(d, _):
            vacc = jnp.zeros((16,), jnp.float32)
            for i in range(_IPW):
                vacc = vacc + jnp.where(
                    masks[i], chunks_v[i, d, pl.ds(offs[i], 16)], 0.0)
            acc_v[d, :] = vacc
            return 0

        lax.fori_loop(0, EMBED_DIM, body, 0)
        pltpu.sync_copy(acc_v, out_hbm.at[wid])


DEPTH = 4
NBLK = NR


def _tc_body(sum_ref, w1_ref, b1_ref, b2_ref, w2_hbm, out_ref,
             buf, h_ref, sems):
    for d in range(DEPTH):
        pltpu.make_async_copy(
            w2_hbm.at[pl.ds(d * VB, VB), :], buf.at[d], sems.at[d]).start()

    bag = jnp.sum(sum_ref[...], axis=0)
    tmp = lax.dot_general(
        bag, w1_ref[...],
        (((0,), (0,)), ((), ())),
        preferred_element_type=jnp.float32,
    )
    pre = jnp.sum(tmp, axis=0, keepdims=True) + b1_ref[...]
    h_ref[...] = jnp.maximum(pre, 0.0)

    def superstep(jd, _):
        base = jd * DEPTH
        for d in range(DEPTH):
            j = base + d
            pltpu.make_async_copy(
                w2_hbm.at[pl.ds(j * VB, VB), :], buf.at[d], sems.at[d]).wait()
            logits = lax.dot_general(
                h_ref[...], buf[d],
                (((1,), (1,)), ((), ())),
                precision=lax.Precision.DEFAULT,
                preferred_element_type=jnp.float32,
            ) + b2_ref[pl.ds(j, 1), :]
            out_ref[pl.ds(j, 1), :] = logits
            nj = j + DEPTH

            @pl.when(nj < NBLK)
            def _():
                pltpu.make_async_copy(
                    w2_hbm.at[pl.ds(nj * VB, VB), :], buf.at[d],
                    sems.at[d]).start()
        return 0

    lax.fori_loop(0, NBLK // DEPTH, superstep, 0)

    x = out_ref[...]
    m = jnp.max(x)
    lse = m + jnp.log(jnp.sum(jnp.exp(x - m)))
    out_ref[...] = x - lse


_tc_call = pl.pallas_call(
    _tc_body,
    in_specs=[
        pl.BlockSpec(memory_space=pltpu.VMEM),
        pl.BlockSpec(memory_space=pltpu.VMEM),
        pl.BlockSpec(memory_space=pltpu.VMEM),
        pl.BlockSpec(memory_space=pltpu.VMEM),
        pl.BlockSpec(memory_space=pltpu.HBM),
    ],
    out_specs=pl.BlockSpec(memory_space=pltpu.VMEM),
    out_shape=jax.ShapeDtypeStruct((NBLK, VB), jnp.float32),
    scratch_shapes=[
        pltpu.VMEM((DEPTH, VB, HIDDEN), jnp.float32),
        pltpu.VMEM((1, HIDDEN), jnp.float32),
        pltpu.SemaphoreType.DMA((DEPTH,)),
    ],
)


def kernel(inputs, emb, W1, b1, W2, b2):
    idx = jnp.concatenate(
        [inputs.astype(jnp.int32).reshape(_NWRK, _IPW),
         jnp.zeros((_NWRK, 16 - _IPW), jnp.int32)], axis=1)
    bag = _sc_bag_sum_kernel()(idx, emb.T)
    b2r = b2.reshape(NBLK, VB)
    out = _tc_call(bag, W1.T, b1.reshape(1, HIDDEN), b2r, W2)
    return out.reshape(1, VOCAB)


# speedup vs baseline: 1.0583x; 1.0583x over previous
import functools

import jax
import jax.numpy as jnp
from jax import lax
from jax.experimental import pallas as pl
from jax.experimental.pallas import tpu as pltpu
from jax.experimental.pallas import tpu_sc as plsc

VOCAB = 100000
EMBED_DIM = 64
HIDDEN = 128
L = 200

NQ = 4
VB = 1000
NB = 25
NR = NQ * NB


_NWRK = 25
_IPW = 8


def _lane_bcast(v, k):
    return lax.gather(
        v, jnp.full((16, 1), k),
        lax.GatherDimensionNumbers(
            offset_dims=(), collapsed_slice_dims=(0,),
            start_index_map=(0,)),
        (1,), mode=lax.GatherScatterMode.PROMISE_IN_BOUNDS)


@functools.cache
def _sc_bag_sum_kernel():
    mesh = plsc.VectorSubcoreMesh(core_axis_name="c", subcore_axis_name="s")
    return pl.kernel(
        _sc_bag_sum,
        out_type=jax.ShapeDtypeStruct((_NWRK, EMBED_DIM, 16), jnp.float32),
        mesh=mesh,
        scratch_types=[
            pltpu.VMEM((16,), jnp.int32),
            pltpu.VMEM((_IPW, EMBED_DIM, 128), jnp.float32),
            pltpu.VMEM((EMBED_DIM, 16), jnp.float32),
            pltpu.SemaphoreType.DMA,
        ],
    )


def _sc_bag_sum(idx_hbm, embT_hbm, out_hbm, idx_v, chunks_v, acc_v, sem):
    cid = lax.axis_index("c")
    sid = lax.axis_index("s")
    wid = sid * 2 + cid

    @pl.when(wid < _NWRK)
    def _():
        pltpu.sync_copy(idx_hbm.at[wid], idx_v)
        vrow = idx_v[...]
        bcs = [_lane_bcast(vrow, i) for i in range(_IPW)]
        cps = []
        for i in range(_IPW):
            col = pl.multiple_of((vrow[i] >> 7) << 7, 128)
            cp = pltpu.make_async_copy(
                embT_hbm.at[:, pl.ds(col, 128)], chunks_v.at[i], sem)
            cp.start()
            cps.append(cp)

        lane = lax.iota(jnp.int32, 16)
        masks = [lane == (bcs[i] & 15) for i in range(_IPW)]
        offs = [((vrow[i] & 127) >> 4) << 4 for i in range(_IPW)]
        for cp in cps:
            cp.wait()

        def body(d, _):
            vacc = jnp.zeros((16,), jnp.float32)
            for i in range(_IPW):
                vacc = vacc + jnp.where(
                    masks[i], chunks_v[i, d, pl.ds(offs[i], 16)], 0.0)
            acc_v[d, :] = vacc
            return 0

        lax.fori_loop(0, EMBED_DIM, body, 0)
        pltpu.sync_copy(acc_v, out_hbm.at[wid])


def _tc_body(sum_ref, w1_ref, b1_ref, *refs):
    w2_refs = refs[:NQ]
    b2_refs = refs[NQ:2 * NQ]
    out_ref = refs[2 * NQ]
    h_ref = refs[2 * NQ + 1]
    j = pl.program_id(0)

    @pl.when(j == 0)
    def _():
        bag = jnp.sum(sum_ref[...], axis=0)
        tmp = lax.dot_general(
            bag, w1_ref[...],
            (((0,), (0,)), ((), ())),
            preferred_element_type=jnp.float32,
        )
        pre = jnp.sum(tmp, axis=0, keepdims=True) + b1_ref[...]
        h_ref[...] = jnp.maximum(pre, 0.0)

    for q in range(NQ):
        logits = lax.dot_general(
            h_ref[...], w2_refs[q][...],
            (((1,), (1,)), ((), ())),
            precision=lax.Precision.DEFAULT,
            preferred_element_type=jnp.float32,
        ) + b2_refs[q][0]
        out_ref[pl.ds(q * NB + j, 1), :] = logits

    @pl.when(j == NB - 1)
    def _():
        x = out_ref[...]
        m = jnp.max(x)
        lse = m + jnp.log(jnp.sum(jnp.exp(x - m)))
        out_ref[...] = x - lse


def _w2_spec(q):
    return pl.BlockSpec((VB, HIDDEN), lambda j, q=q: (q * NB + j, 0))


def _b2_spec(q):
    return pl.BlockSpec((1, 1, VB), lambda j, q=q: (q * NB + j, 0, 0))


_tc_call = pl.pallas_call(
    _tc_body,
    grid=(NB,),
    in_specs=[
        pl.BlockSpec((_NWRK, EMBED_DIM, 16), lambda j: (0, 0, 0)),
        pl.BlockSpec((EMBED_DIM, HIDDEN), lambda j: (0, 0)),
        pl.BlockSpec((1, HIDDEN), lambda j: (0, 0)),
    ] + [_w2_spec(q) for q in range(NQ)] + [_b2_spec(q) for q in range(NQ)],
    out_specs=pl.BlockSpec((NR, VB), lambda j: (0, 0)),
    out_shape=jax.ShapeDtypeStruct((NR, VB), jnp.float32),
    scratch_shapes=[pltpu.VMEM((1, HIDDEN), jnp.float32)],
    compiler_params=pltpu.CompilerParams(
        dimension_semantics=("arbitrary",),
    ),
)


def kernel(inputs, emb, W1, b1, W2, b2):
    idx = jnp.concatenate(
        [inputs.astype(jnp.int32).reshape(_NWRK, _IPW),
         jnp.zeros((_NWRK, 16 - _IPW), jnp.int32)], axis=1)
    bag = _sc_bag_sum_kernel()(idx, emb.T)
    b2r = b2.reshape(NR, 1, VB)
    out = _tc_call(
        bag, W1.T, b1.reshape(1, HIDDEN),
        *([W2] * NQ), *([b2r] * NQ),
    )
    return out.reshape(1, VOCAB)


# speedup vs baseline: 1.2763x; 1.2061x over previous
import functools

import jax
import jax.numpy as jnp
from jax import lax
from jax.experimental import pallas as pl
from jax.experimental.pallas import tpu as pltpu
from jax.experimental.pallas import tpu_sc as plsc

VOCAB = 100000
EMBED_DIM = 64
HIDDEN = 128
L = 200

NQ = 4
VB = 5000
NB = 5
NR = NQ * NB


_NWRK = 25
_IPW = 8


def _lane_bcast(v, k):
    return lax.gather(
        v, jnp.full((16, 1), k),
        lax.GatherDimensionNumbers(
            offset_dims=(), collapsed_slice_dims=(0,),
            start_index_map=(0,)),
        (1,), mode=lax.GatherScatterMode.PROMISE_IN_BOUNDS)


@functools.cache
def _sc_bag_sum_kernel():
    mesh = plsc.VectorSubcoreMesh(core_axis_name="c", subcore_axis_name="s")
    return pl.kernel(
        _sc_bag_sum,
        out_type=jax.ShapeDtypeStruct((_NWRK, EMBED_DIM, 16), jnp.float32),
        mesh=mesh,
        scratch_types=[
            pltpu.VMEM((16,), jnp.int32),
            pltpu.VMEM((_IPW, EMBED_DIM, 128), jnp.float32),
            pltpu.VMEM((EMBED_DIM, 16), jnp.float32),
            pltpu.SemaphoreType.DMA,
        ],
    )


def _sc_bag_sum(idx_hbm, embT_hbm, out_hbm, idx_v, chunks_v, acc_v, sem):
    cid = lax.axis_index("c")
    sid = lax.axis_index("s")
    wid = sid * 2 + cid

    @pl.when(wid < _NWRK)
    def _():
        pltpu.sync_copy(idx_hbm.at[wid], idx_v)
        vrow = idx_v[...]
        bcs = [_lane_bcast(vrow, i) for i in range(_IPW)]
        cps = []
        for i in range(_IPW):
            col = pl.multiple_of((vrow[i] >> 7) << 7, 128)
            cp = pltpu.make_async_copy(
                embT_hbm.at[:, pl.ds(col, 128)], chunks_v.at[i], sem)
            cp.start()
            cps.append(cp)

        lane = lax.iota(jnp.int32, 16)
        masks = [lane == (bcs[i] & 15) for i in range(_IPW)]
        offs = [((vrow[i] & 127) >> 4) << 4 for i in range(_IPW)]
        for cp in cps:
            cp.wait()

        def body(d, _):
            vacc = jnp.zeros((16,), jnp.float32)
            for i in range(_IPW):
                vacc = vacc + jnp.where(
                    masks[i], chunks_v[i, d, pl.ds(offs[i], 16)], 0.0)
            acc_v[d, :] = vacc
            return 0

        lax.fori_loop(0, EMBED_DIM, body, 0)
        pltpu.sync_copy(acc_v, out_hbm.at[wid])


def _tc_body(sum_ref, w1_ref, b1_ref, *refs):
    w2_refs = refs[:NQ]
    b2_refs = refs[NQ:2 * NQ]
    out_ref = refs[2 * NQ]
    h_ref = refs[2 * NQ + 1]
    j = pl.program_id(0)

    @pl.when(j == 0)
    def _():
        bag = jnp.sum(sum_ref[...], axis=0)
        tmp = lax.dot_general(
            bag, w1_ref[...],
            (((0,), (0,)), ((), ())),
            preferred_element_type=jnp.float32,
        )
        pre = jnp.sum(tmp, axis=0, keepdims=True) + b1_ref[...]
        h_ref[...] = jnp.maximum(pre, 0.0)

    for q in range(NQ):
        logits = lax.dot_general(
            h_ref[...], w2_refs[q][...],
            (((1,), (1,)), ((), ())),
            precision=lax.Precision.DEFAULT,
            preferred_element_type=jnp.float32,
        ) + b2_refs[q][0]
        out_ref[pl.ds(q * NB + j, 1), :] = logits

    @pl.when(j == NB - 1)
    def _():
        x = out_ref[...]
        m = jnp.max(x)
        lse = m + jnp.log(jnp.sum(jnp.exp(x - m)))
        out_ref[...] = x - lse


def _w2_spec(q):
    return pl.BlockSpec((VB, HIDDEN), lambda j, q=q: (q * NB + j, 0))


def _b2_spec(q):
    return pl.BlockSpec((1, 1, VB), lambda j, q=q: (q * NB + j, 0, 0))


_tc_call = pl.pallas_call(
    _tc_body,
    grid=(NB,),
    in_specs=[
        pl.BlockSpec((_NWRK, EMBED_DIM, 16), lambda j: (0, 0, 0)),
        pl.BlockSpec((EMBED_DIM, HIDDEN), lambda j: (0, 0)),
        pl.BlockSpec((1, HIDDEN), lambda j: (0, 0)),
    ] + [_w2_spec(q) for q in range(NQ)] + [_b2_spec(q) for q in range(NQ)],
    out_specs=pl.BlockSpec((NR, VB), lambda j: (0, 0)),
    out_shape=jax.ShapeDtypeStruct((NR, VB), jnp.float32),
    scratch_shapes=[pltpu.VMEM((1, HIDDEN), jnp.float32)],
    compiler_params=pltpu.CompilerParams(
        dimension_semantics=("arbitrary",),
    ),
)


def kernel(inputs, emb, W1, b1, W2, b2):
    idx = jnp.concatenate(
        [inputs.astype(jnp.int32).reshape(_NWRK, _IPW),
         jnp.zeros((_NWRK, 16 - _IPW), jnp.int32)], axis=1)
    bag = _sc_bag_sum_kernel()(idx, emb.T)
    b2r = b2.reshape(NR, 1, VB)
    out = _tc_call(
        bag, W1.T, b1.reshape(1, HIDDEN),
        *([W2] * NQ), *([b2r] * NQ),
    )
    return out.reshape(1, VOCAB)
